# Initial kernel scaffold; baseline (speedup 1.0000x reference)
import jax
import jax.numpy as jnp
from jax.experimental import pallas as pl


def kernel(x, edge_index, edge_type, batch, W1, root1, b1, W2, root2, b2, cls_w, cls_b):
    raise NotImplementedError("write your pallas kernel here")


# speedup vs baseline: 9.8058x; 9.8058x over previous
import functools

import jax
import jax.numpy as jnp
from jax import lax
from jax.experimental import pallas as pl
from jax.experimental.pallas import tpu as pltpu
from jax.experimental.pallas import tpu_sc as plsc

N = 10000
E = 320000
R = 10
DIN = 128
H = 64
G = 64

NC = 2
NS = 16
NW = NC * NS

C = 128
EPAD = 323584
EROWS = EPAD // C
RPW = EROWS // NW
RPT = EROWS // NS
NRP = 100352
NPAD = 10240

BN = 400
NB = N // BN


def _dense_body(x_ref, w_ref, root_ref, b_ref, xw_ref, rt_ref):
    r = pl.program_id(1)
    xb = x_ref[...]
    xw_ref[0] = jnp.dot(xb, w_ref[r], preferred_element_type=jnp.float32)

    @pl.when(r == 0)
    def _():
        rt_ref[...] = (
            jnp.dot(xb, root_ref[...], preferred_element_type=jnp.float32)
            + b_ref[...]
        )


def _dense_call(x, w, root, b, din):
    return pl.pallas_call(
        _dense_body,
        grid=(NB, R),
        in_specs=[
            pl.BlockSpec((BN, din), lambda n, r: (n, 0)),
            pl.BlockSpec((R, din, H), lambda n, r: (0, 0, 0)),
            pl.BlockSpec((din, H), lambda n, r: (0, 0)),
            pl.BlockSpec((1, H), lambda n, r: (0, 0)),
        ],
        out_specs=[
            pl.BlockSpec((1, BN, H), lambda n, r: (r, n, 0)),
            pl.BlockSpec((BN, H), lambda n, r: (n, 0)),
        ],
        out_shape=[
            jax.ShapeDtypeStruct((R, N, H), jnp.float32),
            jax.ShapeDtypeStruct((N, H), jnp.float32),
        ],
    )(x, w, root, b.reshape(1, H))


def _dense2_body(a0_ref, a1_ref, rt1_ref, w_ref, root_ref, b_ref, xw_ref, rt_ref):
    r = pl.program_id(1)
    h = jnp.maximum(a0_ref[...] + a1_ref[...] + rt1_ref[...], 0.0)
    xw_ref[0] = jnp.dot(h, w_ref[r], preferred_element_type=jnp.float32)

    @pl.when(r == 0)
    def _():
        rt_ref[...] = (
            jnp.dot(h, root_ref[...], preferred_element_type=jnp.float32)
            + b_ref[...]
        )


def _dense2_call(a0, a1, rt1, w, root, b):
    return pl.pallas_call(
        _dense2_body,
        grid=(NB, R),
        in_specs=[
            pl.BlockSpec((BN, H), lambda n, r: (n, 0)),
            pl.BlockSpec((BN, H), lambda n, r: (n, 0)),
            pl.BlockSpec((BN, H), lambda n, r: (n, 0)),
            pl.BlockSpec((R, H, H), lambda n, r: (0, 0, 0)),
            pl.BlockSpec((H, H), lambda n, r: (0, 0)),
            pl.BlockSpec((1, H), lambda n, r: (0, 0)),
        ],
        out_specs=[
            pl.BlockSpec((1, BN, H), lambda n, r: (r, n, 0)),
            pl.BlockSpec((BN, H), lambda n, r: (n, 0)),
        ],
        out_shape=[
            jax.ShapeDtypeStruct((R, N, H), jnp.float32),
            jax.ShapeDtypeStruct((N, H), jnp.float32),
        ],
    )(a0, a1, rt1, w, root, b.reshape(1, H))


def _head_body(a0_ref, a1_ref, rt2_ref, batch_ref, clsw_ref, clsb_ref,
               out_ref, psum, cnt):
    i = pl.program_id(0)

    @pl.when(i == 0)
    def _():
        psum[...] = jnp.zeros((G, H), jnp.float32)
        cnt[...] = jnp.zeros((G, 1), jnp.float32)

    h = jnp.maximum(a0_ref[...] + a1_ref[...] + rt2_ref[...], 0.0)
    onehot = (batch_ref[...] == lax.broadcasted_iota(jnp.int32, (BN, G), 1))
    onehot = onehot.astype(jnp.float32)
    psum[...] += lax.dot_general(
        onehot, h, (((0,), (0,)), ((), ())),
        preferred_element_type=jnp.float32)
    cnt[...] += lax.dot_general(
        onehot, jnp.ones((BN, 1), jnp.float32), (((0,), (0,)), ((), ())),
        preferred_element_type=jnp.float32)

    @pl.when(i == NB - 1)
    def _():
        mat = jnp.dot(psum[...], clsw_ref[...],
                      preferred_element_type=jnp.float32)
        out_ref[...] = mat / jnp.maximum(cnt[...], 1.0) + clsb_ref[...]


def _head_call(a0, a1, rt2, batch, cls_w, cls_b):
    return pl.pallas_call(
        _head_body,
        grid=(NB,),
        in_specs=[
            pl.BlockSpec((BN, H), lambda n: (n, 0)),
            pl.BlockSpec((BN, H), lambda n: (n, 0)),
            pl.BlockSpec((BN, H), lambda n: (n, 0)),
            pl.BlockSpec((BN, 1), lambda n: (n, 0)),
            pl.BlockSpec((H, 1), lambda n: (0, 0)),
            pl.BlockSpec((1, 1), lambda n: (0, 0)),
        ],
        out_specs=pl.BlockSpec((G, 1), lambda n: (0, 0)),
        out_shape=jax.ShapeDtypeStruct((G, 1), jnp.float32),
        scratch_shapes=[
            pltpu.VMEM((G, H), jnp.float32),
            pltpu.VMEM((G, 1), jnp.float32),
        ],
    )(a0, a1, rt2, batch.reshape(N, 1), cls_w, cls_b.reshape(1, 1))


def _counts_body(src_hbm, dst_hbm, et_hbm,
                 norm_hbm, gidx_hbm, comp_hbm,
                 cnt_sh, sv, dv, ev, comp_v, gidx_v, ones_v, zbuf, cbuf):
    c = lax.axis_index("c")
    s = lax.axis_index("s")

    ZT = NRP // NS

    def zb(i, _):
        zbuf[pl.ds(i * 16, 16)] = jnp.zeros((16,), jnp.float32)
        return _

    lax.fori_loop(0, ZT // 16, zb, None)
    pltpu.sync_copy(zbuf, cnt_sh.at[pl.ds(s * ZT, ZT)])

    def ob(i, _):
        ones_v[pl.ds(i * 16, 16)] = jnp.ones((16,), jnp.float32)
        return _

    lax.fori_loop(0, C // 16, ob, None)
    plsc.subcore_barrier()

    half = RPT // 2

    def chunk(k, _):
        row = s * RPT + k
        pltpu.sync_copy(src_hbm.at[row], sv)
        pltpu.sync_copy(dst_hbm.at[row], dv)
        pltpu.sync_copy(et_hbm.at[row], ev)

        def sub(j, _):
            sl = pl.ds(j * 16, 16)
            et = ev[sl] * N
            comp_v[sl] = et + dv[sl]
            gidx_v[sl] = et + sv[sl]
            return _

        lax.fori_loop(0, C // 16, sub, None)

        own = jnp.where(c == 0, k < half, k >= half)

        @pl.when(own)
        def _():
            pltpu.sync_copy(comp_v, comp_hbm.at[row])
            pltpu.sync_copy(gidx_v, gidx_hbm.at[row])

        pltpu.sync_copy(ones_v, cnt_sh.at[comp_v], add=True)
        return _

    lax.fori_loop(0, RPT, chunk, None)
    plsc.subcore_barrier()

    HS = NRP // NC
    TS = HS // NS
    off = c * HS + s * TS
    pltpu.sync_copy(cnt_sh.at[pl.ds(off, TS)], cbuf)

    def nb(i, _):
        sl = pl.ds(i * 16, 16)
        idx = off + i * 16 + lax.iota(jnp.int32, 16)
        nv = 1.0 / jnp.maximum(cbuf[sl], 1.0)
        cbuf[sl] = jnp.where(idx < N * R, nv, 0.0)
        return _

    lax.fori_loop(0, TS // 16, nb, None)
    pltpu.sync_copy(cbuf, norm_hbm.at[pl.ds(off, TS)])


def _counts_call(src, dst, et):
    return pl.kernel(
        _counts_body,
        out_type=[
            jax.ShapeDtypeStruct((NRP,), jnp.float32),
            jax.ShapeDtypeStruct((EROWS, C), jnp.int32),
            jax.ShapeDtypeStruct((EROWS, C), jnp.int32),
        ],
        mesh=plsc.VectorSubcoreMesh(core_axis_name="c", subcore_axis_name="s"),
        scratch_types=[
            pltpu.VMEM_SHARED((NRP,), jnp.float32),
            pltpu.VMEM((C,), jnp.int32),
            pltpu.VMEM((C,), jnp.int32),
            pltpu.VMEM((C,), jnp.int32),
            pltpu.VMEM((C,), jnp.int32),
            pltpu.VMEM((C,), jnp.int32),
            pltpu.VMEM((C,), jnp.float32),
            pltpu.VMEM((NRP // NS,), jnp.float32),
            pltpu.VMEM((NRP // NC // NS,), jnp.float32),
        ],
    )(src, dst, et)


def _agg_body(xw_hbm, gidx_hbm, comp_hbm, dst_hbm, norm_hbm,
              out_hbm, acc_sh, gv, cv, dv, nv, rows_v, sem, sem2):
    c = lax.axis_index("c")
    s = lax.axis_index("s")
    w = s * NC + c

    def zb(e, _):
        for j in range(4):
            rows_v[e, pl.ds(j * 16, 16)] = jnp.zeros((16,), jnp.float32)
        return _

    lax.fori_loop(0, C, zb, None)
    DT = NPAD // NS
    for j in range(DT // C):
        pltpu.sync_copy(rows_v, acc_sh.at[pl.ds(s * DT + j * C, C)])
    plsc.subcore_barrier()

    def chunk(k, _):
        row = w * RPW + k
        pltpu.sync_copy(gidx_hbm.at[row], gv)
        pltpu.sync_copy(comp_hbm.at[row], cv)
        pltpu.sync_copy(dst_hbm.at[row], dv)
        pltpu.async_copy(norm_hbm.at[cv], nv, sem).wait()
        pltpu.async_copy(xw_hbm.at[gv], rows_v, sem2).wait()

        def sc(g, _):
            nvec = nv[pl.ds(g * 16, 16)]
            for t in range(16):
                ns = nvec[t]
                e = g * 16 + t
                for j in range(4):
                    sl = pl.ds(j * 16, 16)
                    rows_v[e, sl] = rows_v[e, sl] * ns
            return _

        lax.fori_loop(0, C // 16, sc, None)
        pltpu.sync_copy(rows_v, acc_sh.at[dv], add=True)
        return _

    lax.fori_loop(0, RPW, chunk, None)
    plsc.subcore_barrier()

    DT = NPAD // NS
    pltpu.sync_copy(acc_sh.at[pl.ds(s * DT, DT)],
                    out_hbm.at[c, pl.ds(s * DT, DT)])


def _agg_call(xw, gidx, comp, dst, norm):
    return pl.kernel(
        _agg_body,
        out_type=jax.ShapeDtypeStruct((NC, NPAD, H), jnp.float32),
        mesh=plsc.VectorSubcoreMesh(core_axis_name="c", subcore_axis_name="s"),
        scratch_types=[
            pltpu.VMEM_SHARED((NPAD, H), jnp.float32),
            pltpu.VMEM((C,), jnp.int32),
            pltpu.VMEM((C,), jnp.int32),
            pltpu.VMEM((C,), jnp.int32),
            pltpu.VMEM((C,), jnp.float32),
            pltpu.VMEM((C, H), jnp.float32),
            pltpu.SemaphoreType.DMA,
            pltpu.SemaphoreType.DMA,
        ],
        compiler_params=pltpu.CompilerParams(use_tc_tiling_on_sc=False),
    )(xw, gidx, comp, dst, norm)


def kernel(x, edge_index, edge_type, batch, W1, root1, b1, W2, root2, b2,
           cls_w, cls_b):
    pad = EPAD - E
    src = jnp.concatenate(
        [edge_index[0], jnp.full((pad,), -R * N, jnp.int32)]).reshape(EROWS, C)
    dst = jnp.concatenate(
        [edge_index[1], jnp.zeros((pad,), jnp.int32)]).reshape(EROWS, C)
    et = jnp.concatenate(
        [edge_type, jnp.full((pad,), R, jnp.int32)]).reshape(EROWS, C)

    norm, gidx, comp = _counts_call(src, dst, et)
    xw1, rt1 = _dense_call(x, W1, root1, b1, DIN)

    agg1 = _agg_call(xw1.reshape(R * N, H), gidx, comp, dst, norm)
    xw2, rt2 = _dense2_call(agg1[0, :N], agg1[1, :N], rt1, W2, root2, b2)

    agg2 = _agg_call(xw2.reshape(R * N, H), gidx, comp, dst, norm)
    out = _head_call(agg2[0, :N], agg2[1, :N], rt2, batch, cls_w, cls_b)
    return out.reshape(-1)
